# baseline (speedup 1.0000x reference)
import functools

import jax
import jax.numpy as jnp
from jax import lax
from jax.experimental import pallas as pl
from jax.experimental.pallas import tpu as pltpu
from jax.experimental.pallas import tpu_sc as plsc

B = 16384
K = 64
D = 512
N = 100000
NC, NS = 2, 16
NW = NC * NS
BPW = B // NW
CHF = 64
NCHF = BPW // CHF
RPT = K // NW
CHI = 4096
NCHI = B // CHI
GV = 16
UNR = 16

_mesh = plsc.VectorSubcoreMesh(core_axis_name="c", subcore_axis_name="s")


@functools.partial(
    pl.kernel,
    out_type=jax.ShapeDtypeStruct((B, D), jnp.float32),
    mesh=_mesh,
    scratch_types=[
        pltpu.VMEM((BPW,), jnp.int32),
        pltpu.VMEM((CHF, D), jnp.float32),
        pltpu.VMEM((CHF, D), jnp.float32),
        pltpu.SemaphoreType.DMA,
        pltpu.SemaphoreType.DMA,
        pltpu.SemaphoreType.DMA,
        pltpu.SemaphoreType.DMA,
    ],
)
def _sc_gather_f(items_hbm, f_hbm, fe_out, idx_v, b0, b1, si0, si1, so0, so1):
    wid = lax.axis_index("s") * NC + lax.axis_index("c")
    base = wid * BPW
    pltpu.sync_copy(items_hbm.at[pl.ds(base, BPW)], idx_v)
    bufs = (b0, b1)
    sin = (si0, si1)
    sout = (so0, so1)

    def issue_in(c):
        return pltpu.async_copy(
            f_hbm.at[idx_v.at[pl.ds(c * CHF, CHF)]], bufs[c & 1], sin[c & 1])

    def issue_out(c):
        return pltpu.async_copy(
            bufs[c & 1], fe_out.at[pl.ds(base + c * CHF, CHF)], sout[c & 1])

    copies_in = {0: issue_in(0), 1: issue_in(1)}
    copies_out = {}
    for c in range(NCHF):
        copies_in[c].wait()
        copies_out[c] = issue_out(c)
        if c + 2 < NCHF:
            copies_out[c].wait()
            copies_in[c + 2] = issue_in(c + 2)
    copies_out[NCHF - 2].wait()
    copies_out[NCHF - 1].wait()


@functools.partial(
    pl.kernel,
    out_type=(
        jax.ShapeDtypeStruct((K, B), jnp.float32),
        jax.ShapeDtypeStruct((K, B), jnp.float32),
        jax.ShapeDtypeStruct((K, B), jnp.float32),
    ),
    mesh=_mesh,
    scratch_types=[
        pltpu.VMEM((1, N), jnp.float32),
        pltpu.VMEM((B,), jnp.int32),
        pltpu.VMEM((1, CHI), jnp.float32),
        pltpu.VMEM((1, CHI), jnp.float32),
        pltpu.SemaphoreType.DMA,
        pltpu.SemaphoreType.DMA,
    ],
    compiler_params=pltpu.CompilerParams(needs_layout_passes=False),
)
def _sc_gather_t(users_hbm, items_hbm, gu_t_hbm, gi_t_hbm, tu_t_hbm,
                 gu_out, gi_out, tu_out, row_v, idx_v, o0, o1, s0, s1):
    wid = lax.axis_index("s") * NC + lax.axis_index("c")
    zero16 = jnp.zeros((GV,), jnp.int32)
    obufs = (o0, o1)
    osems = (s0, s1)

    def do_row(tab, out, d, pending):
        pltpu.sync_copy(tab.at[pl.ds(d, 1), :], row_v)
        for c in range(NCHI):
            out_v = obufs[c & 1]
            if pending[c & 1] is not None:
                pending[c & 1].wait()

            def body(j, carry):
                for u in range(UNR):
                    o = j * GV * UNR + u * GV
                    iv = idx_v[pl.ds(c * CHI + o, GV)]
                    out_v[0, pl.ds(o, GV)] = plsc.load_gather(
                        row_v, [zero16, iv])
                return carry

            lax.fori_loop(0, CHI // (GV * UNR), body, 0)
            pending[c & 1] = pltpu.async_copy(
                out_v, out.at[pl.ds(d, 1), pl.ds(c * CHI, CHI)],
                osems[c & 1])
        return pending

    pending = [None, None]
    for idx_src, rows in (
        (users_hbm, ((gu_t_hbm, gu_out), (tu_t_hbm, tu_out))),
        (items_hbm, ((gi_t_hbm, gi_out),)),
    ):
        pltpu.sync_copy(idx_src, idx_v)
        for tab, out in rows:
            for r in range(RPT):
                pending = do_row(tab, out, wid * RPT + r, pending)
    for p in pending:
        if p is not None:
            p.wait()


RB = 512
XB = 2048


def _tc_norm_body(fe_ref, wt_ref, b_ref, pn_ref):
    proj_t = lax.dot_general(
        wt_ref[...], fe_ref[...], (((1,), (1,)), ((), ())),
        preferred_element_type=jnp.float32) + b_ref[...]
    ss = jnp.sum(proj_t * proj_t, axis=0, keepdims=True)
    inv = 1.0 / jnp.maximum(jnp.sqrt(ss), 1e-12)
    pn_ref[...] = proj_t * inv


def _tc_norm(effe_i, w_t, proj_b):
    return pl.pallas_call(
        _tc_norm_body,
        grid=(B // RB,),
        in_specs=[
            pl.BlockSpec((RB, D), lambda i: (i, 0)),
            pl.BlockSpec((K, D), lambda i: (0, 0)),
            pl.BlockSpec((K, 1), lambda i: (0, 0)),
        ],
        out_specs=pl.BlockSpec((K, RB), lambda i: (0, i)),
        out_shape=jax.ShapeDtypeStruct((K, B), jnp.float32),
    )(effe_i, w_t, proj_b.reshape(K, 1))


def _tc_xui_body(gut_ref, git_ref, tut_ref, pn_ref, xui_ref):
    xui_ref[...] = (
        jnp.sum(gut_ref[...] * git_ref[...], axis=0, keepdims=True)
        + jnp.sum(tut_ref[...] * pn_ref[...], axis=0, keepdims=True))


def _tc_xui(gu_t, gi_t, tu_t, pn_t):
    xui = pl.pallas_call(
        _tc_xui_body,
        grid=(B // XB,),
        in_specs=[pl.BlockSpec((K, XB), lambda i: (0, i))] * 4,
        out_specs=pl.BlockSpec((1, XB), lambda i: (0, i)),
        out_shape=jax.ShapeDtypeStruct((1, B), jnp.float32),
    )(gu_t, gi_t, tu_t, pn_t)
    return xui.reshape(B)


def kernel(users, items, Gu, Gi, Tu, F, proj_W, proj_b):
    effe_i = _sc_gather_f(items, F)
    pn_t = _tc_norm(effe_i, proj_W.T, proj_b)
    gu_t, gi_t, tu_t = _sc_gather_t(users, items, Gu.T, Gi.T, Tu.T)
    xui = _tc_xui(gu_t, gi_t, tu_t, pn_t)
    return (xui, gu_t.T, gi_t.T, tu_t.T, pn_t.T)


# speedup vs baseline: 1.0298x; 1.0298x over previous
import functools

import jax
import jax.numpy as jnp
from jax import lax
from jax.experimental import pallas as pl
from jax.experimental.pallas import tpu as pltpu
from jax.experimental.pallas import tpu_sc as plsc

B = 16384
K = 64
D = 512
N = 100000
NC, NS = 2, 16
NW = NC * NS
BPW = B // NW
CHF = 64
NCHF = BPW // CHF
RPT = K // NW
CHI = 4096
NCHI = B // CHI
GV = 16
UNR = 8

_mesh = plsc.VectorSubcoreMesh(core_axis_name="c", subcore_axis_name="s")


@functools.partial(
    pl.kernel,
    out_type=jax.ShapeDtypeStruct((B, D), jnp.float32),
    mesh=_mesh,
    scratch_types=[
        pltpu.VMEM((BPW,), jnp.int32),
        pltpu.VMEM((CHF, D), jnp.float32),
        pltpu.VMEM((CHF, D), jnp.float32),
        pltpu.SemaphoreType.DMA,
        pltpu.SemaphoreType.DMA,
        pltpu.SemaphoreType.DMA,
        pltpu.SemaphoreType.DMA,
    ],
)
def _sc_gather_f(items_hbm, f_hbm, fe_out, idx_v, b0, b1, si0, si1, so0, so1):
    wid = lax.axis_index("s") * NC + lax.axis_index("c")
    base = wid * BPW
    pltpu.sync_copy(items_hbm.at[pl.ds(base, BPW)], idx_v)
    bufs = (b0, b1)
    sin = (si0, si1)
    sout = (so0, so1)

    def issue_in(c):
        return pltpu.async_copy(
            f_hbm.at[idx_v.at[pl.ds(c * CHF, CHF)]], bufs[c & 1], sin[c & 1])

    def issue_out(c):
        return pltpu.async_copy(
            bufs[c & 1], fe_out.at[pl.ds(base + c * CHF, CHF)], sout[c & 1])

    copies_in = {0: issue_in(0), 1: issue_in(1)}
    copies_out = {}
    for c in range(NCHF):
        copies_in[c].wait()
        copies_out[c] = issue_out(c)
        if c + 2 < NCHF:
            copies_out[c].wait()
            copies_in[c + 2] = issue_in(c + 2)
    copies_out[NCHF - 2].wait()
    copies_out[NCHF - 1].wait()


@functools.partial(
    pl.kernel,
    out_type=(
        jax.ShapeDtypeStruct((K, B), jnp.float32),
        jax.ShapeDtypeStruct((K, B), jnp.float32),
        jax.ShapeDtypeStruct((K, B), jnp.float32),
    ),
    mesh=_mesh,
    scratch_types=[
        pltpu.VMEM((1, N), jnp.float32),
        pltpu.VMEM((B,), jnp.int32),
        pltpu.VMEM((1, CHI), jnp.float32),
        pltpu.VMEM((1, CHI), jnp.float32),
        pltpu.SemaphoreType.DMA,
        pltpu.SemaphoreType.DMA,
    ],
    compiler_params=pltpu.CompilerParams(needs_layout_passes=False),
)
def _sc_gather_t(users_hbm, items_hbm, gu_t_hbm, gi_t_hbm, tu_t_hbm,
                 gu_out, gi_out, tu_out, row_v, idx_v, o0, o1, s0, s1):
    wid = lax.axis_index("s") * NC + lax.axis_index("c")
    zero16 = jnp.zeros((GV,), jnp.int32)
    obufs = (o0, o1)
    osems = (s0, s1)

    def do_row(tab, out, d, pending):
        pltpu.sync_copy(tab.at[pl.ds(d, 1), :], row_v)
        for c in range(NCHI):
            out_v = obufs[c & 1]
            if pending[c & 1] is not None:
                pending[c & 1].wait()

            def body(j, carry):
                for u in range(UNR):
                    o = j * GV * UNR + u * GV
                    iv = idx_v[pl.ds(c * CHI + o, GV)]
                    out_v[0, pl.ds(o, GV)] = plsc.load_gather(
                        row_v, [zero16, iv])
                return carry

            lax.fori_loop(0, CHI // (GV * UNR), body, 0)
            pending[c & 1] = pltpu.async_copy(
                out_v, out.at[pl.ds(d, 1), pl.ds(c * CHI, CHI)],
                osems[c & 1])
        return pending

    pending = [None, None]
    for idx_src, rows in (
        (users_hbm, ((gu_t_hbm, gu_out), (tu_t_hbm, tu_out))),
        (items_hbm, ((gi_t_hbm, gi_out),)),
    ):
        pltpu.sync_copy(idx_src, idx_v)
        for tab, out in rows:
            for r in range(RPT):
                pending = do_row(tab, out, wid * RPT + r, pending)
    for p in pending:
        if p is not None:
            p.wait()


RB = 512
XB = 2048


def _tc_norm_body(fe_ref, wt_ref, b_ref, pn_ref):
    proj_t = lax.dot_general(
        wt_ref[...], fe_ref[...], (((1,), (1,)), ((), ())),
        preferred_element_type=jnp.float32) + b_ref[...]
    ss = jnp.sum(proj_t * proj_t, axis=0, keepdims=True)
    inv = 1.0 / jnp.maximum(jnp.sqrt(ss), 1e-12)
    pn_ref[...] = proj_t * inv


def _tc_norm(effe_i, w_t, proj_b):
    return pl.pallas_call(
        _tc_norm_body,
        grid=(B // RB,),
        in_specs=[
            pl.BlockSpec((RB, D), lambda i: (i, 0)),
            pl.BlockSpec((K, D), lambda i: (0, 0)),
            pl.BlockSpec((K, 1), lambda i: (0, 0)),
        ],
        out_specs=pl.BlockSpec((K, RB), lambda i: (0, i)),
        out_shape=jax.ShapeDtypeStruct((K, B), jnp.float32),
    )(effe_i, w_t, proj_b.reshape(K, 1))


def _tc_xui_body(gut_ref, git_ref, tut_ref, pn_ref, xui_ref):
    xui_ref[...] = (
        jnp.sum(gut_ref[...] * git_ref[...], axis=0, keepdims=True)
        + jnp.sum(tut_ref[...] * pn_ref[...], axis=0, keepdims=True))


def _tc_xui(gu_t, gi_t, tu_t, pn_t):
    xui = pl.pallas_call(
        _tc_xui_body,
        grid=(B // XB,),
        in_specs=[pl.BlockSpec((K, XB), lambda i: (0, i))] * 4,
        out_specs=pl.BlockSpec((1, XB), lambda i: (0, i)),
        out_shape=jax.ShapeDtypeStruct((1, B), jnp.float32),
    )(gu_t, gi_t, tu_t, pn_t)
    return xui.reshape(B)


def kernel(users, items, Gu, Gi, Tu, F, proj_W, proj_b):
    effe_i = _sc_gather_f(items, F)
    pn_t = _tc_norm(effe_i, proj_W.T, proj_b)
    gu_t, gi_t, tu_t = _sc_gather_t(users, items, Gu.T, Gi.T, Tu.T)
    xui = _tc_xui(gu_t, gi_t, tu_t, pn_t)
    return (xui, gu_t.T, gi_t.T, tu_t.T, pn_t.T)
